# baseline (speedup 1.0000x reference)
import functools

import numpy as np
import jax
import jax.numpy as jnp
from jax import lax
from jax.experimental import pallas as pl
from jax.experimental.pallas import tpu as pltpu
from jax.experimental.pallas import tpu_sc as plsc

_NCLS = 100
_B = 1024
_NN = 99
_SCALE = 2.0 / (_B * _NN)
_NW = 32
_SPW = _B // _NW
_D = 8


def _build_tree(num_classes):
    nodes = []

    def rec(leaves):
        if len(leaves) <= 1:
            return
        mid = len(leaves) // 2
        nodes.append((leaves[:mid], leaves[mid:]))
        rec(leaves[:mid])
        rec(leaves[mid:])

    rec(list(range(num_classes)))
    return nodes


def _build_tables():
    nodes = _build_tree(_NCLS)
    lrow = [L[0] if len(L) == 1 else 100 + (i + 1) for i, (L, R) in enumerate(nodes)]
    rrow = [R[0] if len(R) == 1 else 100 + (i + len(L)) for i, (L, R) in enumerate(nodes)]
    ti = np.zeros((2, _NCLS, _D), np.int32)
    tf = np.zeros((4, _NCLS, _D), np.float32)
    for t in range(_NCLS):
        i, d = 0, 0
        while True:
            L, R = nodes[i]
            child = 0 if t in L else 1
            ti[0, t, d] = lrow[i]
            ti[1, t, d] = rrow[i]
            tf[0, t, d] = 1.0 / len(L)
            tf[1, t, d] = 1.0 / len(R)
            tf[2 + child, t, d] = _SCALE
            d += 1
            sub = L if child == 0 else R
            if len(sub) == 1:
                break
            i = (i + 1) if child == 0 else (i + len(L))
    return lrow, rrow, ti.reshape(-1), tf.reshape(-1)


_LROW, _RROW, _TI, _TF = _build_tables()
_T8 = _NCLS * _D


@functools.partial(
    pl.kernel,
    out_type=jax.ShapeDtypeStruct((_NW, 16), jnp.float32),
    mesh=plsc.VectorSubcoreMesh(core_axis_name="c", subcore_axis_name="s"),
    compiler_params=pltpu.CompilerParams(needs_layout_passes=False),
    scratch_types=[
        pltpu.VMEM((200 * _SPW,), jnp.float32),
        pltpu.VMEM((2 * _T8,), jnp.int32),
        pltpu.VMEM((4 * _T8,), jnp.float32),
        pltpu.VMEM((_SPW,), jnp.int32),
        pltpu.VMEM((16,), jnp.float32),
    ],
)
def _sc_loss(x_hbm, t_hbm, ti_hbm, tf_hbm, out_hbm, vals, ti_v, tf_v, tg_v, acc_v):
    wid = lax.axis_index("s") * 2 + lax.axis_index("c")
    pltpu.sync_copy(x_hbm.at[wid], vals.at[pl.ds(0, _NCLS * _SPW)])
    pltpu.sync_copy(t_hbm.at[pl.ds(wid * _SPW, _SPW)], tg_v)
    pltpu.sync_copy(ti_hbm, ti_v)
    pltpu.sync_copy(tf_hbm, tf_v)

    for g in range(2):
        base = g * 16
        for i in range(_NN - 1, -1, -1):
            lo = _LROW[i] * _SPW + base
            ro = _RROW[i] * _SPW + base
            vals[pl.ds((100 + i) * _SPW + base, 16)] = (
                vals[pl.ds(lo, 16)] + vals[pl.ds(ro, 16)]
            )

    lane = lax.iota(jnp.int32, 16)
    total = jnp.zeros((16,), jnp.float32)
    for g in range(2):
        col = lane + g * 16
        t = tg_v[pl.ds(g * 16, 16)]
        idx0 = t * _D
        acc = jnp.zeros((16,), jnp.float32)
        for d in range(_D):
            idx = idx0 + d
            lr = plsc.load_gather(ti_v, [idx])
            rr = plsc.load_gather(ti_v, [idx + _T8])
            r0 = plsc.load_gather(tf_v, [idx])
            r1 = plsc.load_gather(tf_v, [idx + _T8])
            w0 = plsc.load_gather(tf_v, [idx + 2 * _T8])
            w1 = plsc.load_gather(tf_v, [idx + 3 * _T8])
            vl = plsc.load_gather(vals, [lr * _SPW + col])
            vr = plsc.load_gather(vals, [rr * _SPW + col])
            m0 = vl * r0
            m1 = vr * r1
            mx = jnp.maximum(m0, m1)
            z = jnp.exp(-jnp.abs(m0 - m1))
            u = z / (z + np.float32(2.0))
            u2 = u * u
            q = u2 * np.float32(2.0 / 9.0) + np.float32(2.0 / 7.0)
            q = q * u2 + np.float32(2.0 / 5.0)
            q = q * u2 + np.float32(2.0 / 3.0)
            q = q * u2 + np.float32(2.0)
            lse = mx + q * u
            acc = acc + (w0 + w1) * lse - w0 * m0 - w1 * m1
        total = total + acc
    acc_v[...] = total
    pltpu.sync_copy(acc_v, out_hbm.at[wid])


def kernel(outputs, targets):
    x = (
        outputs.astype(jnp.float32)
        .reshape(_NW, _SPW, _NCLS)
        .transpose(0, 2, 1)
        .reshape(_NW, _NCLS * _SPW)
    )
    t = targets.astype(jnp.int32)
    out = _sc_loss(x, t, jnp.asarray(_TI), jnp.asarray(_TF))
    return jnp.sum(out)


# speedup vs baseline: 1.0367x; 1.0367x over previous
import functools

import numpy as np
import jax
import jax.numpy as jnp
from jax import lax
from jax.experimental import pallas as pl
from jax.experimental.pallas import tpu as pltpu
from jax.experimental.pallas import tpu_sc as plsc

_NCLS = 100
_B = 1024
_NN = 99
_SCALE = 2.0 / (_B * _NN)
_NW = 32
_SPW = _B // _NW
_D = 8


def _build_tree(num_classes):
    nodes = []

    def rec(leaves):
        if len(leaves) <= 1:
            return
        mid = len(leaves) // 2
        nodes.append((leaves[:mid], leaves[mid:]))
        rec(leaves[:mid])
        rec(leaves[mid:])

    rec(list(range(num_classes)))
    return nodes


def _build_tables():
    nodes = _build_tree(_NCLS)
    lrow = [L[0] if len(L) == 1 else 100 + (i + 1) for i, (L, R) in enumerate(nodes)]
    rrow = [R[0] if len(R) == 1 else 100 + (i + len(L)) for i, (L, R) in enumerate(nodes)]
    tp = np.zeros((_NCLS, _D), np.int32)
    tp[:, :] = (1 << 16) | (1 << 22)
    for t in range(_NCLS):
        i, d = 0, 0
        while True:
            L, R = nodes[i]
            child = 0 if t in L else 1
            tp[t, d] = (
                lrow[i]
                | (rrow[i] << 8)
                | (len(L) << 16)
                | (len(R) << 22)
                | (child << 28)
                | (1 << 29)
            )
            d += 1
            sub = L if child == 0 else R
            if len(sub) == 1:
                break
            i = (i + 1) if child == 0 else (i + len(L))
    return lrow, rrow, tp.reshape(-1)


_LROW, _RROW, _TP = _build_tables()


@functools.partial(
    pl.kernel,
    out_type=jax.ShapeDtypeStruct((_NW, 16), jnp.float32),
    mesh=plsc.VectorSubcoreMesh(core_axis_name="c", subcore_axis_name="s"),
    compiler_params=pltpu.CompilerParams(needs_layout_passes=False),
    scratch_types=[
        pltpu.VMEM((_SPW * _NCLS,), jnp.float32),
        pltpu.VMEM((200 * _SPW,), jnp.float32),
        pltpu.VMEM((_NCLS * _D,), jnp.int32),
        pltpu.VMEM((_SPW,), jnp.int32),
        pltpu.VMEM((16,), jnp.float32),
        pltpu.SemaphoreType.DMA,
        pltpu.SemaphoreType.DMA,
        pltpu.SemaphoreType.DMA,
    ],
)
def _sc_loss(x_hbm, t_hbm, tp_hbm, out_hbm, xs_v, vals, tp_v, tg_v, acc_v,
             sem_x, sem_t, sem_p):
    wid = lax.axis_index("s") * 2 + lax.axis_index("c")
    cp_x = pltpu.async_copy(
        x_hbm.at[pl.ds(wid * (_SPW * _NCLS), _SPW * _NCLS)], xs_v, sem_x
    )
    cp_t = pltpu.async_copy(t_hbm.at[pl.ds(wid * _SPW, _SPW)], tg_v, sem_t)
    cp_p = pltpu.async_copy(tp_hbm, tp_v, sem_p)
    cp_x.wait()

    lane = lax.iota(jnp.int32, 16)

    for g in range(2):
        base_idx = lane * _NCLS + g * (16 * _NCLS)
        for c in range(_NCLS):
            vals[pl.ds(c * _SPW + g * 16, 16)] = plsc.load_gather(
                xs_v, [base_idx + c]
            )

    for g in range(2):
        base = g * 16
        for i in range(_NN - 1, -1, -1):
            lo = _LROW[i] * _SPW + base
            ro = _RROW[i] * _SPW + base
            vals[pl.ds((100 + i) * _SPW + base, 16)] = (
                vals[pl.ds(lo, 16)] + vals[pl.ds(ro, 16)]
            )

    cp_t.wait()
    cp_p.wait()
    total = jnp.zeros((16,), jnp.float32)
    for g in range(2):
        col = lane + g * 16
        t = tg_v[pl.ds(g * 16, 16)]
        idx0 = t * _D
        acc = jnp.zeros((16,), jnp.float32)
        for d in range(_D):
            p = plsc.load_gather(tp_v, [idx0 + d])
            lr = p & 255
            rr = lax.shift_right_logical(p, 8) & 255
            sl = lax.shift_right_logical(p, 16) & 63
            sr = lax.shift_right_logical(p, 22) & 63
            cbit = lax.shift_right_logical(p, 28) & 1
            vbit = lax.shift_right_logical(p, 29) & 1
            vl = plsc.load_gather(vals, [lr * _SPW + col])
            vr = plsc.load_gather(vals, [rr * _SPW + col])
            m0 = vl / sl.astype(jnp.float32)
            m1 = vr / sr.astype(jnp.float32)
            mx = jnp.maximum(m0, m1)
            z = jnp.exp(-jnp.abs(m0 - m1))
            u = z / (z + np.float32(2.0))
            u2 = u * u
            q = u2 * np.float32(2.0 / 9.0) + np.float32(2.0 / 7.0)
            q = q * u2 + np.float32(2.0 / 5.0)
            q = q * u2 + np.float32(2.0 / 3.0)
            q = q * u2 + np.float32(2.0)
            lse = mx + q * u
            wv = vbit.astype(jnp.float32) * np.float32(_SCALE)
            w1 = wv * cbit.astype(jnp.float32)
            w0 = wv - w1
            acc = acc + wv * lse - w0 * m0 - w1 * m1
        total = total + acc
    acc_v[...] = total
    pltpu.sync_copy(acc_v, out_hbm.at[wid])


def kernel(outputs, targets):
    x = outputs.astype(jnp.float32).reshape(_B * _NCLS)
    t = targets.astype(jnp.int32)
    out = _sc_loss(x, t, jnp.asarray(_TP))
    return jnp.sum(out)


# speedup vs baseline: 1.1045x; 1.0655x over previous
import functools

import numpy as np
import jax
import jax.numpy as jnp
from jax import lax
from jax.experimental import pallas as pl
from jax.experimental.pallas import tpu as pltpu
from jax.experimental.pallas import tpu_sc as plsc

_NCLS = 100
_B = 1024
_NN = 99
_SCALE = 2.0 / (_B * _NN)
_NW = 32
_SPW = _B // _NW
_D = 8


def _build_tree(num_classes):
    nodes = []

    def rec(leaves):
        if len(leaves) <= 1:
            return
        mid = len(leaves) // 2
        nodes.append((leaves[:mid], leaves[mid:]))
        rec(leaves[:mid])
        rec(leaves[mid:])

    rec(list(range(num_classes)))
    return nodes


def _build_tables():
    nodes = _build_tree(_NCLS)
    lrow = [L[0] if len(L) == 1 else 100 + (i + 1) for i, (L, R) in enumerate(nodes)]
    rrow = [R[0] if len(R) == 1 else 100 + (i + len(L)) for i, (L, R) in enumerate(nodes)]
    tp = np.zeros((_NCLS, _D), np.int32)
    tp[:, :] = (1 << 16) | (1 << 22)
    for t in range(_NCLS):
        i, d = 0, 0
        while True:
            L, R = nodes[i]
            child = 0 if t in L else 1
            tp[t, d] = (
                lrow[i]
                | (rrow[i] << 8)
                | (len(L) << 16)
                | (len(R) << 22)
                | (child << 28)
                | (1 << 29)
            )
            d += 1
            sub = L if child == 0 else R
            if len(sub) == 1:
                break
            i = (i + 1) if child == 0 else (i + len(L))
    return lrow, rrow, tp.reshape(-1)


_LROW, _RROW, _TP = _build_tables()


@functools.partial(
    pl.kernel,
    out_type=jax.ShapeDtypeStruct((_NW, 16), jnp.float32),
    mesh=plsc.VectorSubcoreMesh(core_axis_name="c", subcore_axis_name="s"),
    compiler_params=pltpu.CompilerParams(needs_layout_passes=False),
    scratch_types=[
        pltpu.VMEM((200 * _SPW,), jnp.float32),
        pltpu.VMEM((_NCLS * _D,), jnp.int32),
        pltpu.VMEM((_SPW,), jnp.int32),
        pltpu.VMEM((16,), jnp.float32),
        pltpu.SemaphoreType.DMA,
        pltpu.SemaphoreType.DMA,
        pltpu.SemaphoreType.DMA,
    ],
)
def _sc_loss(x_hbm, t_hbm, tp_hbm, out_hbm, vals, tp_v, tg_v, acc_v,
             sem_x, sem_t, sem_p):
    wid = lax.axis_index("s") * 2 + lax.axis_index("c")
    cp_x = pltpu.async_copy(
        x_hbm.at[pl.ds(wid * (_SPW * _NCLS), _SPW * _NCLS)],
        vals.at[pl.ds(0, _NCLS * _SPW)],
        sem_x,
    )
    cp_t = pltpu.async_copy(t_hbm.at[pl.ds(wid * _SPW, _SPW)], tg_v, sem_t)
    cp_p = pltpu.async_copy(tp_hbm, tp_v, sem_p)
    cp_x.wait()

    lane = lax.iota(jnp.int32, 16)

    for g in range(2):
        base = g * 16
        for i in range(_NN - 1, -1, -1):
            lo = _LROW[i] * _SPW + base
            ro = _RROW[i] * _SPW + base
            vals[pl.ds((100 + i) * _SPW + base, 16)] = (
                vals[pl.ds(lo, 16)] + vals[pl.ds(ro, 16)]
            )

    cp_t.wait()
    cp_p.wait()
    total = jnp.zeros((16,), jnp.float32)
    for g in range(2):
        col = lane + g * 16
        t = tg_v[pl.ds(g * 16, 16)]
        idx0 = t * _D
        acc = jnp.zeros((16,), jnp.float32)
        for d in range(_D):
            p = plsc.load_gather(tp_v, [idx0 + d])
            lr = p & 255
            rr = lax.shift_right_logical(p, 8) & 255
            sl = lax.shift_right_logical(p, 16) & 63
            sr = lax.shift_right_logical(p, 22) & 63
            cbit = lax.shift_right_logical(p, 28) & 1
            vbit = lax.shift_right_logical(p, 29) & 1
            vl = plsc.load_gather(vals, [lr * _SPW + col])
            vr = plsc.load_gather(vals, [rr * _SPW + col])
            m0 = vl / sl.astype(jnp.float32)
            m1 = vr / sr.astype(jnp.float32)
            mx = jnp.maximum(m0, m1)
            z = jnp.exp(-jnp.abs(m0 - m1))
            u = z / (z + np.float32(2.0))
            u2 = u * u
            q = u2 * np.float32(2.0 / 9.0) + np.float32(2.0 / 7.0)
            q = q * u2 + np.float32(2.0 / 5.0)
            q = q * u2 + np.float32(2.0 / 3.0)
            q = q * u2 + np.float32(2.0)
            lse = mx + q * u
            wv = vbit.astype(jnp.float32) * np.float32(_SCALE)
            w1 = wv * cbit.astype(jnp.float32)
            w0 = wv - w1
            acc = acc + wv * lse - w0 * m0 - w1 * m1
        total = total + acc
    acc_v[...] = total
    pltpu.sync_copy(acc_v, out_hbm.at[wid])


def kernel(outputs, targets):
    x = (
        outputs.astype(jnp.float32)
        .reshape(_NW, _SPW, _NCLS)
        .transpose(0, 2, 1)
        .reshape(_NW * _NCLS * _SPW)
    )
    t = targets.astype(jnp.int32)
    out = _sc_loss(x, t, jnp.asarray(_TP))
    return jnp.sum(out)


# speedup vs baseline: 1.1064x; 1.0017x over previous
import functools

import numpy as np
import jax
import jax.numpy as jnp
from jax import lax
from jax.experimental import pallas as pl
from jax.experimental.pallas import tpu as pltpu
from jax.experimental.pallas import tpu_sc as plsc

_NCLS = 100
_B = 1024
_NN = 99
_SCALE = 2.0 / (_B * _NN)
_NW = 32
_SPW = _B // _NW
_D = 7


def _build_tree(num_classes):
    nodes = []

    def rec(leaves):
        if len(leaves) <= 1:
            return
        mid = len(leaves) // 2
        nodes.append((leaves[:mid], leaves[mid:]))
        rec(leaves[:mid])
        rec(leaves[mid:])

    rec(list(range(num_classes)))
    return nodes


def _build_tables():
    nodes = _build_tree(_NCLS)
    lrow = [L[0] if len(L) == 1 else 100 + (i + 1) for i, (L, R) in enumerate(nodes)]
    rrow = [R[0] if len(R) == 1 else 100 + (i + len(L)) for i, (L, R) in enumerate(nodes)]
    tp = np.zeros((_NCLS, _D), np.int32)
    tp[:, :] = (1 << 16) | (1 << 22)
    for t in range(_NCLS):
        i, d = 0, 0
        while True:
            L, R = nodes[i]
            child = 0 if t in L else 1
            tp[t, d] = (
                lrow[i]
                | (rrow[i] << 8)
                | (len(L) << 16)
                | (len(R) << 22)
                | (child << 28)
                | (1 << 29)
            )
            d += 1
            sub = L if child == 0 else R
            if len(sub) == 1:
                break
            i = (i + 1) if child == 0 else (i + len(L))
    return lrow, rrow, tp.reshape(-1)


_LROW, _RROW, _TP = _build_tables()


@functools.partial(
    pl.kernel,
    out_type=jax.ShapeDtypeStruct((_NW, 16), jnp.float32),
    mesh=plsc.VectorSubcoreMesh(core_axis_name="c", subcore_axis_name="s"),
    compiler_params=pltpu.CompilerParams(needs_layout_passes=False),
    scratch_types=[
        pltpu.VMEM((200 * _SPW,), jnp.float32),
        pltpu.VMEM((_NCLS * _D,), jnp.int32),
        pltpu.VMEM((_SPW,), jnp.int32),
        pltpu.VMEM((16,), jnp.float32),
        pltpu.SemaphoreType.DMA,
        pltpu.SemaphoreType.DMA,
        pltpu.SemaphoreType.DMA,
    ],
)
def _sc_loss(x_hbm, t_hbm, tp_hbm, out_hbm, vals, tp_v, tg_v, acc_v,
             sem_x, sem_t, sem_p):
    wid = lax.axis_index("s") * 2 + lax.axis_index("c")
    cp_x = pltpu.async_copy(
        x_hbm.at[pl.ds(wid * (_SPW * _NCLS), _SPW * _NCLS)],
        vals.at[pl.ds(0, _NCLS * _SPW)],
        sem_x,
    )
    cp_t = pltpu.async_copy(t_hbm.at[pl.ds(wid * _SPW, _SPW)], tg_v, sem_t)
    cp_p = pltpu.async_copy(tp_hbm, tp_v, sem_p)
    cp_x.wait()

    lane = lax.iota(jnp.int32, 16)

    for g in range(2):
        base = g * 16
        for i in range(_NN - 1, 0, -1):
            lo = _LROW[i] * _SPW + base
            ro = _RROW[i] * _SPW + base
            vals[pl.ds((100 + i) * _SPW + base, 16)] = (
                vals[pl.ds(lo, 16)] + vals[pl.ds(ro, 16)]
            )

    cp_t.wait()
    cp_p.wait()
    total = jnp.zeros((16,), jnp.float32)
    for g in range(2):
        col = lane + g * 16
        t = tg_v[pl.ds(g * 16, 16)]
        idx0 = t * _D
        acc = jnp.zeros((16,), jnp.float32)
        for d in range(_D):
            p = plsc.load_gather(tp_v, [idx0 + d])
            lr = p & 255
            rr = lax.shift_right_logical(p, 8) & 255
            sl = lax.shift_right_logical(p, 16) & 63
            sr = lax.shift_right_logical(p, 22) & 63
            cbit = lax.shift_right_logical(p, 28) & 1
            vbit = lax.shift_right_logical(p, 29) & 1
            vl = plsc.load_gather(vals, [lr * _SPW + col])
            vr = plsc.load_gather(vals, [rr * _SPW + col])
            m0 = vl / sl.astype(jnp.float32)
            m1 = vr / sr.astype(jnp.float32)
            mx = jnp.maximum(m0, m1)
            z = jnp.exp(-jnp.abs(m0 - m1))
            u = z / (z + np.float32(2.0))
            u2 = u * u
            q = u2 * np.float32(2.0 / 9.0) + np.float32(2.0 / 7.0)
            q = q * u2 + np.float32(2.0 / 5.0)
            q = q * u2 + np.float32(2.0 / 3.0)
            q = q * u2 + np.float32(2.0)
            lse = mx + q * u
            if d < 6:
                wv = jnp.full((16,), np.float32(_SCALE), jnp.float32)
            else:
                wv = vbit.astype(jnp.float32) * np.float32(_SCALE)
            w1 = wv * cbit.astype(jnp.float32)
            w0 = wv - w1
            acc = acc + wv * lse - w0 * m0 - w1 * m1
        total = total + acc
    acc_v[...] = total
    pltpu.sync_copy(acc_v, out_hbm.at[wid])


def kernel(outputs, targets):
    x = (
        outputs.astype(jnp.float32)
        .reshape(_NW, _SPW, _NCLS)
        .transpose(0, 2, 1)
        .reshape(_NW * _NCLS * _SPW)
    )
    t = targets.astype(jnp.int32)
    out = _sc_loss(x, t, jnp.asarray(_TP))
    return jnp.sum(out)
